# baseline (speedup 1.0000x reference)
import functools

import jax
import jax.numpy as jnp
from jax import lax
from jax.experimental import pallas as pl
from jax.experimental.pallas import tpu as pltpu
from jax.experimental.pallas import tpu_sc as plsc

N = 10000
E = 160000
D = 256
H = 128
NC = 2
NS = 16
NW = NC * NS

CHUNK = 56
NBUF = 4
E_PAD = 161280
CH_PER_TILE = E_PAD // NS // CHUNK
E_PER_TILE_C = CH_PER_TILE * CHUNK
E_PER_TILE_A = E_PAD // NW
VECS_A = E_PER_TILE_A // 16
ROWS_PER_TILE = N // NS
ACC_ROWS = N + 16
HIST = ACC_ROWS

_mesh = plsc.VectorSubcoreMesh(core_axis_name="c", subcore_axis_name="s")
_sc_params = pltpu.CompilerParams(
    needs_layout_passes=False, use_tc_tiling_on_sc=False)


@functools.partial(
    pl.kernel,
    out_type=jax.ShapeDtypeStruct((NW, HIST), jnp.float32),
    mesh=_mesh,
    scratch_types=[
        pltpu.VMEM((E_PER_TILE_A,), jnp.int32),
        pltpu.VMEM((HIST,), jnp.float32),
    ],
    compiler_params=_sc_params,
)
def _deg_kernel(dst_hbm, out_hbm, ids_v, hist_v):
    c = lax.axis_index("c")
    s = lax.axis_index("s")
    wid = c * NS + s
    pltpu.sync_copy(dst_hbm.at[wid], ids_v)
    zeros = jnp.zeros((16,), jnp.float32)

    def zbody(i, _):
        hist_v[pl.ds(i * 16, 16)] = zeros
        return 0

    lax.fori_loop(0, HIST // 16, zbody, 0)
    ones = jnp.ones((16,), jnp.float32)

    def abody(j, _):
        idx = ids_v[pl.ds(j * 16, 16)]
        plsc.addupdate_scatter(hist_v, [idx], ones)
        return 0

    lax.fori_loop(0, VECS_A, abody, 0)
    pltpu.sync_copy(hist_v, out_hbm.at[wid])


def _h_body(x_ref, w_ref, degp_ref, h_ref):
    deg = 1.0 + jnp.sum(degp_ref[...], axis=1)
    dis = lax.rsqrt(deg)
    h = jnp.dot(x_ref[...], w_ref[...], preferred_element_type=jnp.float32)
    hp = h * dis[:, None]
    h_ref[...] = jnp.stack([hp[:, :H], hp[:, H:]])


def _h_call(x, w, deg_parts):
    R = 2000
    grid = (N // R,)
    return pl.pallas_call(
        _h_body,
        grid=grid,
        in_specs=[
            pl.BlockSpec((R, D), lambda i: (i, 0)),
            pl.BlockSpec((D, D), lambda i: (0, 0)),
            pl.BlockSpec((R, NW), lambda i: (i, 0)),
        ],
        out_specs=pl.BlockSpec((NC, R, H), lambda i: (0, i, 0)),
        out_shape=jax.ShapeDtypeStruct((NC, N, H), jnp.float32),
    )(x, w, deg_parts)


@functools.partial(
    pl.kernel,
    out_type=jax.ShapeDtypeStruct((NC, N, H), jnp.float32),
    mesh=_mesh,
    scratch_types=[
        pltpu.VMEM_SHARED((ACC_ROWS, H), jnp.float32),
        pltpu.VMEM((CH_PER_TILE, CHUNK), jnp.int32),
        pltpu.VMEM((CH_PER_TILE, CHUNK), jnp.int32),
    ] + [pltpu.VMEM((CHUNK, H), jnp.float32) for _ in range(NBUF)]
      + [pltpu.SemaphoreType.DMA for _ in range(NBUF)],
    compiler_params=_sc_params,
)
def _agg_kernel(h_hbm, src_hbm, dst_hbm, out_hbm,
                acc_sh, src_v, dst_v, *bufs_and_sems):
    bufs = bufs_and_sems[:NBUF]
    sems = bufs_and_sems[NBUF:]
    c = lax.axis_index("c")
    s = lax.axis_index("s")
    pltpu.sync_copy(src_hbm.at[s], src_v)
    pltpu.sync_copy(dst_hbm.at[s], dst_v)

    def gather(j, t):
        pltpu.async_copy(h_hbm.at[c].at[src_v.at[j]], bufs[t], sems[t])

    for t in range(NBUF):
        gather(t, t)
    pltpu.sync_copy(h_hbm.at[c].at[pl.ds(s * ROWS_PER_TILE, ROWS_PER_TILE)],
                    acc_sh.at[pl.ds(s * ROWS_PER_TILE, ROWS_PER_TILE)])
    plsc.subcore_barrier()

    def loop_i(i, _):
        for t in range(NBUF):
            j = i * NBUF + t
            pltpu.make_async_copy(h_hbm.at[c].at[src_v.at[j]],
                                  bufs[t], sems[t]).wait()
            pltpu.sync_copy(bufs[t], acc_sh.at[dst_v.at[j]], add=True)
            gather(j + NBUF, t)
        return 0

    lax.fori_loop(0, CH_PER_TILE // NBUF - 1, loop_i, 0)
    for t in range(NBUF):
        j = CH_PER_TILE - NBUF + t
        pltpu.make_async_copy(h_hbm.at[c].at[src_v.at[j]],
                              bufs[t], sems[t]).wait()
        pltpu.sync_copy(bufs[t], acc_sh.at[dst_v.at[j]], add=True)
    plsc.subcore_barrier()
    pltpu.sync_copy(acc_sh.at[pl.ds(s * ROWS_PER_TILE, ROWS_PER_TILE)],
                    out_hbm.at[c].at[pl.ds(s * ROWS_PER_TILE, ROWS_PER_TILE)])


def _out_body(pre_ref, degp_ref, b_ref, o_ref):
    deg = 1.0 + jnp.sum(degp_ref[...], axis=1)
    dis = lax.rsqrt(deg)
    merged = jnp.concatenate([pre_ref[0], pre_ref[1]], axis=1)
    o_ref[...] = merged * dis[:, None] + b_ref[...]


def _out_call(pre, deg_parts, b):
    R = 2000
    grid = (N // R,)
    return pl.pallas_call(
        _out_body,
        grid=grid,
        in_specs=[
            pl.BlockSpec((NC, R, H), lambda i: (0, i, 0)),
            pl.BlockSpec((R, NW), lambda i: (i, 0)),
            pl.BlockSpec((1, D), lambda i: (0, 0)),
        ],
        out_specs=pl.BlockSpec((R, D), lambda i: (i, 0)),
        out_shape=jax.ShapeDtypeStruct((N, D), jnp.float32),
    )(pre, deg_parts, b)


def kernel(x, edge_index, W, b):
    ei = edge_index.astype(jnp.int32)
    pad = E_PAD - E
    src = jnp.concatenate([ei[0], jnp.zeros((pad,), jnp.int32)])
    dst = jnp.concatenate([ei[1], jnp.full((pad,), N, jnp.int32)])

    dst_a = dst.reshape(NW, E_PER_TILE_A)
    src_t = src.reshape(NS, CH_PER_TILE, CHUNK)
    dst_t = dst.reshape(NS, CH_PER_TILE, CHUNK)

    deg_parts = _deg_kernel(dst_a)
    degp = deg_parts[:, :N].T
    hp = _h_call(x, W, degp)
    pre = _agg_kernel(hp, src_t, dst_t)
    out = _out_call(pre, degp, b.reshape(1, D))
    return out


# speedup vs baseline: 1.0026x; 1.0026x over previous
import functools

import jax
import jax.numpy as jnp
from jax import lax
from jax.experimental import pallas as pl
from jax.experimental.pallas import tpu as pltpu
from jax.experimental.pallas import tpu_sc as plsc

N = 10000
E = 160000
D = 256
H = 128
NC = 2
NS = 16
NW = NC * NS

CHUNK = 56
NBUF = 4
E_PAD = 161280
CH_PER_TILE = E_PAD // NS // CHUNK
E_PER_TILE_C = CH_PER_TILE * CHUNK
E_PER_TILE_A = E_PAD // NW
VECS_A = E_PER_TILE_A // 16
ROWS_PER_TILE = N // NS
ACC_ROWS = N + 16
HIST = ACC_ROWS

_mesh = plsc.VectorSubcoreMesh(core_axis_name="c", subcore_axis_name="s")
_sc_params = pltpu.CompilerParams(
    needs_layout_passes=False, use_tc_tiling_on_sc=False)


@functools.partial(
    pl.kernel,
    out_type=jax.ShapeDtypeStruct((NW, HIST), jnp.float32),
    mesh=_mesh,
    scratch_types=[
        pltpu.VMEM((E_PER_TILE_A,), jnp.int32),
        pltpu.VMEM((HIST,), jnp.float32),
        pltpu.SemaphoreType.DMA,
    ],
    compiler_params=_sc_params,
)
def _deg_kernel(dst_hbm, out_hbm, ids_v, hist_v, sem):
    c = lax.axis_index("c")
    s = lax.axis_index("s")
    wid = c * NS + s
    pltpu.async_copy(dst_hbm.at[wid], ids_v, sem)
    zeros = jnp.zeros((16,), jnp.float32)

    def zbody(i, _):
        hist_v[pl.ds(i * 16, 16)] = zeros
        return 0

    lax.fori_loop(0, HIST // 16, zbody, 0)
    pltpu.make_async_copy(dst_hbm.at[wid], ids_v, sem).wait()
    ones = jnp.ones((16,), jnp.float32)

    def abody(j, _):
        idx = ids_v[pl.ds(j * 16, 16)]
        plsc.addupdate_scatter(hist_v, [idx], ones)
        return 0

    lax.fori_loop(0, VECS_A, abody, 0)
    pltpu.sync_copy(hist_v, out_hbm.at[wid])


def _h_body(x_ref, w_ref, degp_ref, h_ref):
    deg = 1.0 + jnp.sum(degp_ref[...], axis=1)
    dis = lax.rsqrt(deg)
    h = jnp.dot(x_ref[...], w_ref[...], preferred_element_type=jnp.float32)
    hp = h * dis[:, None]
    h_ref[...] = jnp.stack([hp[:, :H], hp[:, H:]])


def _h_call(x, w, deg_parts):
    R = 2000
    grid = (N // R,)
    return pl.pallas_call(
        _h_body,
        grid=grid,
        in_specs=[
            pl.BlockSpec((R, D), lambda i: (i, 0)),
            pl.BlockSpec((D, D), lambda i: (0, 0)),
            pl.BlockSpec((R, NW), lambda i: (i, 0)),
        ],
        out_specs=pl.BlockSpec((NC, R, H), lambda i: (0, i, 0)),
        out_shape=jax.ShapeDtypeStruct((NC, N, H), jnp.float32),
    )(x, w, deg_parts)


@functools.partial(
    pl.kernel,
    out_type=jax.ShapeDtypeStruct((NC, N, H), jnp.float32),
    mesh=_mesh,
    scratch_types=[
        pltpu.VMEM_SHARED((ACC_ROWS, H), jnp.float32),
        pltpu.VMEM((CH_PER_TILE, CHUNK), jnp.int32),
        pltpu.VMEM((CH_PER_TILE, CHUNK), jnp.int32),
    ] + [pltpu.VMEM((CHUNK, H), jnp.float32) for _ in range(NBUF)]
      + [pltpu.SemaphoreType.DMA for _ in range(NBUF)],
    compiler_params=_sc_params,
)
def _agg_kernel(h_hbm, src_hbm, dst_hbm, out_hbm,
                acc_sh, src_v, dst_v, *bufs_and_sems):
    bufs = bufs_and_sems[:NBUF]
    sems = bufs_and_sems[NBUF:]
    c = lax.axis_index("c")
    s = lax.axis_index("s")
    pltpu.sync_copy(src_hbm.at[s], src_v)
    pltpu.sync_copy(dst_hbm.at[s], dst_v)

    def gather(j, t):
        pltpu.async_copy(h_hbm.at[c].at[src_v.at[j]], bufs[t], sems[t])

    for t in range(NBUF):
        gather(t, t)
    pltpu.sync_copy(h_hbm.at[c].at[pl.ds(s * ROWS_PER_TILE, ROWS_PER_TILE)],
                    acc_sh.at[pl.ds(s * ROWS_PER_TILE, ROWS_PER_TILE)])
    plsc.subcore_barrier()

    def loop_i(i, _):
        for t in range(NBUF):
            j = i * NBUF + t
            pltpu.make_async_copy(h_hbm.at[c].at[src_v.at[j]],
                                  bufs[t], sems[t]).wait()
            pltpu.sync_copy(bufs[t], acc_sh.at[dst_v.at[j]], add=True)
            gather(j + NBUF, t)
        return 0

    lax.fori_loop(0, CH_PER_TILE // NBUF - 1, loop_i, 0)
    for t in range(NBUF):
        j = CH_PER_TILE - NBUF + t
        pltpu.make_async_copy(h_hbm.at[c].at[src_v.at[j]],
                              bufs[t], sems[t]).wait()
        pltpu.sync_copy(bufs[t], acc_sh.at[dst_v.at[j]], add=True)
    plsc.subcore_barrier()
    pltpu.sync_copy(acc_sh.at[pl.ds(s * ROWS_PER_TILE, ROWS_PER_TILE)],
                    out_hbm.at[c].at[pl.ds(s * ROWS_PER_TILE, ROWS_PER_TILE)])


def _out_body(pre_ref, degp_ref, b_ref, o_ref):
    deg = 1.0 + jnp.sum(degp_ref[...], axis=1)
    dis = lax.rsqrt(deg)
    merged = jnp.concatenate([pre_ref[0], pre_ref[1]], axis=1)
    o_ref[...] = merged * dis[:, None] + b_ref[...]


def _out_call(pre, deg_parts, b):
    R = 2000
    grid = (N // R,)
    return pl.pallas_call(
        _out_body,
        grid=grid,
        in_specs=[
            pl.BlockSpec((NC, R, H), lambda i: (0, i, 0)),
            pl.BlockSpec((R, NW), lambda i: (i, 0)),
            pl.BlockSpec((1, D), lambda i: (0, 0)),
        ],
        out_specs=pl.BlockSpec((R, D), lambda i: (i, 0)),
        out_shape=jax.ShapeDtypeStruct((N, D), jnp.float32),
    )(pre, deg_parts, b)


def kernel(x, edge_index, W, b):
    ei = edge_index.astype(jnp.int32)
    pad = E_PAD - E
    src = jnp.concatenate([ei[0], jnp.zeros((pad,), jnp.int32)])
    dst = jnp.concatenate([ei[1], jnp.full((pad,), N, jnp.int32)])

    dst_a = dst.reshape(NW, E_PER_TILE_A)
    src_t = src.reshape(NS, CH_PER_TILE, CHUNK)
    dst_t = dst.reshape(NS, CH_PER_TILE, CHUNK)

    deg_parts = _deg_kernel(dst_a)
    degp = deg_parts[:, :N].T
    hp = _h_call(x, W, degp)
    pre = _agg_kernel(hp, src_t, dst_t)
    out = _out_call(pre, degp, b.reshape(1, D))
    return out
